# Initial kernel scaffold; baseline (speedup 1.0000x reference)
import jax
import jax.numpy as jnp
from jax.experimental import pallas as pl


def kernel(seq_inputs, e1_pos_inputs, e2_pos_inputs, we, wpe):
    raise NotImplementedError("write your pallas kernel here")


# speedup vs baseline: 3.9580x; 3.9580x over previous
import functools

import jax
import jax.numpy as jnp
from jax import lax
from jax.experimental import pallas as pl
from jax.experimental.pallas import tpu as pltpu
from jax.experimental.pallas import tpu_sc as plsc

NW = 32
T = 256
IB = 128
VL = 16


def _sc_embed(seq_blk, e1_blk, e2_blk, we, wpe, n, dw, dp):
    d = dw + 2 * dp
    per_w = n // NW
    steps = per_w // T
    k = T // IB

    mesh = plsc.VectorSubcoreMesh(core_axis_name="c", subcore_axis_name="s")

    @functools.partial(
        pl.kernel,
        out_type=jax.ShapeDtypeStruct((n, d), jnp.float32),
        mesh=mesh,
        compiler_params=pltpu.CompilerParams(use_tc_tiling_on_sc=False),
        scratch_types=[
            pltpu.VMEM((k, IB), jnp.int32),
            pltpu.VMEM((k, IB), jnp.int32),
            pltpu.VMEM((k, IB), jnp.int32),
            pltpu.VMEM((T, dw), jnp.float32),
            pltpu.VMEM((T, dp), jnp.float32),
            pltpu.VMEM((T, dp), jnp.float32),
            pltpu.VMEM((T, d), jnp.float32),
            pltpu.SemaphoreType.DMA,
        ],
    )
    def body(seq_hbm, e1_hbm, e2_hbm, we_hbm, wpe_hbm, out_hbm,
             si_v, p1_v, p2_v, w_v, q1_v, q2_v, rows_v, sem):
        cid = lax.axis_index("c")
        sid = lax.axis_index("s")
        wid = sid * 2 + cid
        sblk0 = wid * steps

        def step(t, carry):
            row0 = wid * per_w + t * T
            pltpu.sync_copy(seq_hbm.at[sblk0 + t], si_v)
            pltpu.sync_copy(e1_hbm.at[sblk0 + t], p1_v)
            pltpu.sync_copy(e2_hbm.at[sblk0 + t], p2_v)
            copies = []
            for j in range(k):
                r = pl.ds(j * IB, IB)
                copies.append(pltpu.make_async_copy(
                    we_hbm.at[si_v.at[j]], w_v.at[r], sem))
                copies.append(pltpu.make_async_copy(
                    wpe_hbm.at[p1_v.at[j]], q1_v.at[r], sem))
                copies.append(pltpu.make_async_copy(
                    wpe_hbm.at[p2_v.at[j]], q2_v.at[r], sem))
            for c in copies:
                c.start()
            for c in copies:
                c.wait()

            def fuse(r, carry2):
                for u in range(2):
                    row = r * 2 + u
                    for c in range(dw // VL):
                        rows_v[row, pl.ds(c * VL, VL)] = (
                            w_v[row, pl.ds(c * VL, VL)])
                    for c in range(dp // VL):
                        rows_v[row, pl.ds(dw + c * VL, VL)] = (
                            q1_v[row, pl.ds(c * VL, VL)])
                    for c in range(dp // VL):
                        rows_v[row, pl.ds(dw + dp + c * VL, VL)] = (
                            q2_v[row, pl.ds(c * VL, VL)])
                return carry2

            lax.fori_loop(0, T // 2, fuse, 0)
            pltpu.sync_copy(rows_v, out_hbm.at[pl.ds(row0, T)])
            return carry

        lax.fori_loop(0, steps, step, 0)

    return body(seq_blk, e1_blk, e2_blk, we, wpe)


def kernel(seq_inputs, e1_pos_inputs, e2_pos_inputs, we, wpe):
    b, l = seq_inputs.shape
    dw = we.shape[1]
    dp = wpe.shape[1]
    n = b * l
    assert n % (NW * T) == 0 and T % IB == 0
    k = T // IB
    seq_blk = seq_inputs.reshape(n // T, k, IB)
    e1_blk = e1_pos_inputs.reshape(n // T, k, IB)
    e2_blk = e2_pos_inputs.reshape(n // T, k, IB)
    out = _sc_embed(seq_blk, e1_blk, e2_blk, we, wpe, n, dw, dp)
    return out.reshape(b, l, dw + 2 * dp)


# speedup vs baseline: 5.0446x; 1.2745x over previous
import functools

import jax
import jax.numpy as jnp
from jax import lax
from jax.experimental import pallas as pl
from jax.experimental.pallas import tpu as pltpu
from jax.experimental.pallas import tpu_sc as plsc

NW = 32
T = 256
IB = 128
VL = 16


def _sc_embed(seq_blk, e1_blk, e2_blk, we, wpe, n, dw, dp):
    d = dw + 2 * dp
    per_w = n // NW
    steps = per_w // T
    k = T // IB

    mesh = plsc.VectorSubcoreMesh(core_axis_name="c", subcore_axis_name="s")

    @functools.partial(
        pl.kernel,
        out_type=jax.ShapeDtypeStruct((n, d), jnp.float32),
        mesh=mesh,
        compiler_params=pltpu.CompilerParams(use_tc_tiling_on_sc=False),
        scratch_types=[
            pltpu.VMEM((k, IB), jnp.int32),
            pltpu.VMEM((k, IB), jnp.int32),
            pltpu.VMEM((k, IB), jnp.int32),
            pltpu.VMEM((T, dw), jnp.float32),
            pltpu.VMEM((T, dp), jnp.float32),
            pltpu.VMEM((T, dp), jnp.float32),
            pltpu.SemaphoreType.DMA,
        ],
    )
    def body(seq_hbm, e1_hbm, e2_hbm, we_hbm, wpe_hbm, out_hbm,
             si_v, p1_v, p2_v, w_v, q1_v, q2_v, sem):
        cid = lax.axis_index("c")
        sid = lax.axis_index("s")
        wid = sid * 2 + cid
        sblk0 = wid * steps

        def step(t, carry):
            row0 = wid * per_w + t * T
            pltpu.sync_copy(seq_hbm.at[sblk0 + t], si_v)
            pltpu.sync_copy(e1_hbm.at[sblk0 + t], p1_v)
            pltpu.sync_copy(e2_hbm.at[sblk0 + t], p2_v)
            copies = []
            for j in range(k):
                r = pl.ds(j * IB, IB)
                copies.append(pltpu.make_async_copy(
                    we_hbm.at[si_v.at[j]], w_v.at[r], sem))
                copies.append(pltpu.make_async_copy(
                    wpe_hbm.at[p1_v.at[j]], q1_v.at[r], sem))
                copies.append(pltpu.make_async_copy(
                    wpe_hbm.at[p2_v.at[j]], q2_v.at[r], sem))
            for c in copies:
                c.start()
            for c in copies:
                c.wait()

            rows = pl.ds(row0, T)
            outs = [
                pltpu.make_async_copy(
                    w_v, out_hbm.at[rows, pl.ds(0, dw)], sem),
                pltpu.make_async_copy(
                    q1_v, out_hbm.at[rows, pl.ds(dw, dp)], sem),
                pltpu.make_async_copy(
                    q2_v, out_hbm.at[rows, pl.ds(dw + dp, dp)], sem),
            ]
            for c in outs:
                c.start()
            for c in outs:
                c.wait()
            return carry

        lax.fori_loop(0, steps, step, 0)

    return body(seq_blk, e1_blk, e2_blk, we, wpe)


def kernel(seq_inputs, e1_pos_inputs, e2_pos_inputs, we, wpe):
    b, l = seq_inputs.shape
    dw = we.shape[1]
    dp = wpe.shape[1]
    n = b * l
    assert n % (NW * T) == 0 and T % IB == 0
    k = T // IB
    seq_blk = seq_inputs.reshape(n // T, k, IB)
    e1_blk = e1_pos_inputs.reshape(n // T, k, IB)
    e2_blk = e2_pos_inputs.reshape(n // T, k, IB)
    out = _sc_embed(seq_blk, e1_blk, e2_blk, we, wpe, n, dw, dp)
    return out.reshape(b, l, dw + 2 * dp)
